# baseline (speedup 1.0000x reference)
import jax
import jax.numpy as jnp
from jax import lax
from jax.experimental import pallas as pl
from jax.experimental.pallas import tpu as pltpu
from jax.experimental.pallas import tpu_sc as plsc

B = 1024
CTX_DIM = 512
N_CLS_CTX = 4
PRE = 5
SUF = 68
TOK = PRE + N_CLS_CTX + SUF

D_CLS = N_CLS_CTX * CTX_DIM
D_PRE = PRE * CTX_DIM
D_SUF = SUF * CTX_DIM
D_OUT = TOK * CTX_DIM

NC = 2
NS = 16
NW = NC * NS
BPW = B // NW


def _sc_body(label_hbm, table_hbm, pre_hbm, suf_hbm, out_hbm,
             idx_v, cls_v, pre_v, suf_v, gsem):
  wid = lax.axis_index("s") * NC + lax.axis_index("c")
  base = wid * BPW
  pltpu.sync_copy(label_hbm.at[pl.ds(base, BPW)], idx_v)
  gather = pltpu.make_async_copy(table_hbm.at[idx_v], cls_v, gsem)
  gather.start()
  pltpu.sync_copy(pre_hbm, pre_v)
  pltpu.sync_copy(suf_hbm, suf_v)
  gather.wait()

  def row_fn(j, carry):
    row = base + j
    pltpu.sync_copy(pre_v, out_hbm.at[row, pl.ds(0, D_PRE)])
    pltpu.sync_copy(cls_v.at[j], out_hbm.at[row, pl.ds(D_PRE, D_CLS)])
    pltpu.sync_copy(suf_v, out_hbm.at[row, pl.ds(D_PRE + D_CLS, D_SUF)])
    return carry

  lax.fori_loop(0, BPW, row_fn, 0)


@jax.jit
def kernel(label, clsctx, token_prefix, token_suffix):
  table = clsctx.reshape(clsctx.shape[0], D_CLS)
  pre = token_prefix.reshape(D_PRE)
  suf = token_suffix.reshape(D_SUF)
  idx = label.astype(jnp.int32)

  run = pl.kernel(
      _sc_body,
      out_type=jax.ShapeDtypeStruct((B, D_OUT), jnp.float32),
      mesh=plsc.VectorSubcoreMesh(core_axis_name="c", subcore_axis_name="s"),
      scratch_types=[
          pltpu.VMEM((BPW,), jnp.int32),
          pltpu.VMEM((BPW, D_CLS), jnp.float32),
          pltpu.VMEM((D_PRE,), jnp.float32),
          pltpu.VMEM((D_SUF,), jnp.float32),
          pltpu.SemaphoreType.DMA,
      ],
  )
  out = run(idx, table, pre, suf)
  return out.reshape(B, TOK, CTX_DIM)
